# baseline (speedup 1.0000x reference)
import functools

import jax
import jax.numpy as jnp
from jax import lax
from jax.experimental import pallas as pl
from jax.experimental.pallas import tpu as pltpu
from jax.experimental.pallas import tpu_sc as plsc

_D = 1024
_SEG = _D // 8
_NC = 2
_NS = 16
_NW = _NC * _NS


@functools.lru_cache(maxsize=None)
def _make_gather(n_rows: int):
    assert n_rows % (8 * _NW) == 0
    b_per_w = n_rows // _NW
    cr = 32
    ci = cr * 8
    ng = 2
    assert b_per_w % cr == 0
    n_chunks = b_per_w // cr
    nbuf = 3

    mesh = plsc.VectorSubcoreMesh(core_axis_name="c", subcore_axis_name="s")

    @functools.partial(
        pl.kernel,
        mesh=mesh,
        out_type=jax.ShapeDtypeStruct((n_rows * 8, _SEG), jnp.float32),
        scratch_types=[
            pltpu.VMEM((b_per_w,), jnp.int32),
        ]
        + [pltpu.VMEM((ci, _SEG), jnp.float32) for _ in range(nbuf)]
        + [pltpu.VMEM((ci,), jnp.int32) for _ in range(nbuf)]
        + [pltpu.VMEM((32,), jnp.int32)]
        + [pltpu.SemaphoreType.DMA for _ in range(2 * nbuf)],
    )
    def gather(table_hbm, idx_hbm, out_hbm, idx_v, *scratch):
        bufs = scratch[:nbuf]
        idxes = scratch[nbuf:2 * nbuf]
        tmp = scratch[2 * nbuf]
        gsems = scratch[2 * nbuf + 1:3 * nbuf + 1]
        wsems = scratch[3 * nbuf + 1:]
        wid = lax.axis_index("s") * _NC + lax.axis_index("c")
        base = wid * b_per_w
        pltpu.sync_copy(idx_hbm.at[pl.ds(base, b_per_w)], idx_v)

        lane = lax.iota(jnp.int32, 16)
        low = lane < 8

        def expand(chunk, slot):
            for h in range(cr // 16):
                w = idx_v[pl.ds(chunk * cr + h * 16, 16)] * 8
                tmp[pl.ds(0, 16)] = w
                tmp[pl.ds(16, 16)] = w
                sw = tmp[pl.ds(8, 16)]
                dup = (jnp.where(low, w, sw), jnp.where(low, sw, w))
                for v in range(8):
                    t = (2 * v + (lane >> 3)) & 7
                    idxes[slot][pl.ds(h * 128 + v * 16, 16)] = dup[v >> 2] + t

        def start_gather(chunk, slot):
            expand(chunk, slot)
            return [
                pltpu.async_copy(
                    table_hbm.at[idxes[slot].at[pl.ds(g * 128, 128)]],
                    bufs[slot].at[pl.ds(g * 128, 128)],
                    gsems[slot],
                )
                for g in range(ng)
            ]

        gathers = [None] * nbuf
        writes = [None] * nbuf
        for j in range(min(nbuf - 1, n_chunks)):
            gathers[j] = start_gather(j, j)
        for i in range(n_chunks):
            p = i % nbuf
            nx = i + nbuf - 1
            if nx < n_chunks:
                q = nx % nbuf
                if writes[q] is not None:
                    writes[q].wait()
                    writes[q] = None
                gathers[q] = start_gather(nx, q)
            for d in gathers[p]:
                d.wait()
            writes[p] = pltpu.async_copy(
                bufs[p],
                out_hbm.at[pl.ds(base * 8 + i * ci, ci)],
                wsems[p],
            )
        for w in writes:
            if w is not None:
                w.wait()

    return gather


def kernel(x_len, index, r_pos):
    b, l, _ = index.shape
    table = jnp.reshape(r_pos, (r_pos.shape[1] * 8, _SEG))
    idx = jnp.reshape(index, (b * l,)).astype(jnp.int32)
    out = _make_gather(b * l)(table, idx)
    out = jnp.reshape(out, (b * l // 8, 8, 8, _SEG))
    out = jnp.transpose(out, (0, 2, 1, 3))
    return jnp.reshape(out, (b, l, _D))


# speedup vs baseline: 1.0082x; 1.0082x over previous
import functools

import jax
import jax.numpy as jnp
from jax import lax
from jax.experimental import pallas as pl
from jax.experimental.pallas import tpu as pltpu
from jax.experimental.pallas import tpu_sc as plsc

_D = 1024
_SEG = _D // 8
_NC = 2
_NS = 16
_NW = _NC * _NS


@functools.lru_cache(maxsize=None)
def _make_gather(n_rows: int):
    assert n_rows % (8 * _NW) == 0
    b_per_w = n_rows // _NW
    cr = 16
    ci = cr * 8
    ng = 1
    assert b_per_w % cr == 0
    n_chunks = b_per_w // cr
    nbuf = 6

    mesh = plsc.VectorSubcoreMesh(core_axis_name="c", subcore_axis_name="s")

    @functools.partial(
        pl.kernel,
        mesh=mesh,
        out_type=jax.ShapeDtypeStruct((n_rows * 8, _SEG), jnp.float32),
        scratch_types=[
            pltpu.VMEM((b_per_w,), jnp.int32),
        ]
        + [pltpu.VMEM((ci, _SEG), jnp.float32) for _ in range(nbuf)]
        + [pltpu.VMEM((ci,), jnp.int32) for _ in range(nbuf)]
        + [pltpu.VMEM((32,), jnp.int32)]
        + [pltpu.SemaphoreType.DMA for _ in range(2 * nbuf)],
    )
    def gather(table_hbm, idx_hbm, out_hbm, idx_v, *scratch):
        bufs = scratch[:nbuf]
        idxes = scratch[nbuf:2 * nbuf]
        tmp = scratch[2 * nbuf]
        gsems = scratch[2 * nbuf + 1:3 * nbuf + 1]
        wsems = scratch[3 * nbuf + 1:]
        wid = lax.axis_index("s") * _NC + lax.axis_index("c")
        base = wid * b_per_w
        pltpu.sync_copy(idx_hbm.at[pl.ds(base, b_per_w)], idx_v)

        lane = lax.iota(jnp.int32, 16)
        low = lane < 8

        def expand(chunk, slot):
            for h in range(cr // 16):
                w = idx_v[pl.ds(chunk * cr + h * 16, 16)] * 8
                tmp[pl.ds(0, 16)] = w
                tmp[pl.ds(16, 16)] = w
                sw = tmp[pl.ds(8, 16)]
                dup = (jnp.where(low, w, sw), jnp.where(low, sw, w))
                for v in range(8):
                    t = (2 * v + (lane >> 3)) & 7
                    idxes[slot][pl.ds(h * 128 + v * 16, 16)] = dup[v >> 2] + t

        def start_gather(chunk, slot):
            expand(chunk, slot)
            return [
                pltpu.async_copy(
                    table_hbm.at[idxes[slot].at[pl.ds(g * 128, 128)]],
                    bufs[slot].at[pl.ds(g * 128, 128)],
                    gsems[slot],
                )
                for g in range(ng)
            ]

        gathers = [None] * nbuf
        writes = [None] * nbuf
        for j in range(min(nbuf - 1, n_chunks)):
            gathers[j] = start_gather(j, j)
        for i in range(n_chunks):
            p = i % nbuf
            nx = i + nbuf - 1
            if nx < n_chunks:
                q = nx % nbuf
                if writes[q] is not None:
                    writes[q].wait()
                    writes[q] = None
                gathers[q] = start_gather(nx, q)
            for d in gathers[p]:
                d.wait()
            writes[p] = pltpu.async_copy(
                bufs[p],
                out_hbm.at[pl.ds(base * 8 + i * ci, ci)],
                wsems[p],
            )
        for w in writes:
            if w is not None:
                w.wait()

    return gather


def kernel(x_len, index, r_pos):
    b, l, _ = index.shape
    table = jnp.reshape(r_pos, (r_pos.shape[1] * 8, _SEG))
    idx = jnp.reshape(index, (b * l,)).astype(jnp.int32)
    out = _make_gather(b * l)(table, idx)
    out = jnp.reshape(out, (b * l // 8, 8, 8, _SEG))
    out = jnp.transpose(out, (0, 2, 1, 3))
    return jnp.reshape(out, (b, l, _D))
